# Initial kernel scaffold; baseline (speedup 1.0000x reference)
import jax
import jax.numpy as jnp
from jax.experimental import pallas as pl


def kernel(idx, emb1, emb2, W1, b1, W2, b2):
    raise NotImplementedError("write your pallas kernel here")


# speedup vs baseline: 7.8235x; 7.8235x over previous
import jax
import jax.numpy as jnp
from jax.experimental import pallas as pl

_N = 10000
_D = 64
_K = 32
_ALPHA = 3.0
_NP = 10240
_R = 200
_NB = _N // _R


def _mlp_body(e1_ref, e2_ref, w1_ref, b1_ref, w2_ref, b2_ref, n1_ref, n2_ref):
    dn = (((1,), (1,)), ((), ()))
    h1 = jax.lax.dot_general(e1_ref[...], w1_ref[...], dn)
    h2 = jax.lax.dot_general(e2_ref[...], w2_ref[...], dn)
    n1_ref[...] = jnp.tanh(_ALPHA * (h1 + b1_ref[...]))
    n2_ref[...] = jnp.tanh(_ALPHA * (h2 + b2_ref[...]))


def _adj_body(x1_ref, x2_ref, n1_ref, n2_ref, out_ref):
    dn = (((1,), (1,)), ((), ()))
    a = (jax.lax.dot_general(x1_ref[...], n2_ref[...], dn)
         - jax.lax.dot_general(x2_ref[...], n1_ref[...], dn))
    act = jnp.maximum(jnp.tanh(_ALPHA * a), 0.0)
    vi = jax.lax.bitcast_convert_type(act, jnp.int32)

    vmax = jnp.max(vi, axis=1, keepdims=True)
    lo0 = jnp.full_like(vmax, -1)
    nhi0 = jnp.zeros_like(vmax)

    def vstep(_, carry):
        lo, hi, nhi = carry
        mid = lo + jax.lax.div(hi - lo, 2)
        cnt = jnp.sum((vi > mid).astype(jnp.int32), axis=1, keepdims=True)
        ge = cnt >= _K
        return (jnp.where(ge, mid, lo),
                jnp.where(ge, hi, mid),
                jnp.where(ge, nhi, cnt))

    _, thr, ngt = jax.lax.fori_loop(0, 30, vstep, (lo0, vmax, nhi0))

    need = _K - ngt
    eq = vi == thr
    cols = jax.lax.broadcasted_iota(jnp.int32, (_R, _NP), 1)

    def cstep(_, carry):
        lo_c, hi_c = carry
        mid = lo_c + jax.lax.div(hi_c - lo_c, 2)
        cnt = jnp.sum((eq & (cols < mid)).astype(jnp.int32), axis=1,
                      keepdims=True)
        ge = cnt >= need
        return (jnp.where(ge, lo_c, mid), jnp.where(ge, mid, hi_c))

    _, cut = jax.lax.fori_loop(0, 14, cstep,
                               (jnp.zeros_like(vmax),
                                jnp.full_like(vmax, 16384)))

    keep = (vi > thr) | (eq & (cols < cut))
    out_ref[...] = jnp.where(keep, act, 0.0)[:, :_N]


def kernel(idx, emb1, emb2, W1, b1, W2, b2):
    e1 = jnp.take(emb1, idx, axis=0)
    e2 = jnp.take(emb2, idx, axis=0)
    pad = ((0, _NP - _N), (0, 0))
    e1p = jnp.pad(e1, pad)
    e2p = jnp.pad(e2, pad)
    nv_shape = jax.ShapeDtypeStruct((_NP, _D), jnp.float32)
    n1p, n2p = pl.pallas_call(
        _mlp_body,
        out_shape=[nv_shape, nv_shape],
    )(e1p, e2p, W1, b1.reshape(1, _D), W2, b2.reshape(1, _D))

    row_spec = pl.BlockSpec((_R, _D), lambda i: (i, 0))
    full_spec = pl.BlockSpec((_NP, _D), lambda i: (0, 0))
    adj = pl.pallas_call(
        _adj_body,
        grid=(_NB,),
        in_specs=[row_spec, row_spec, full_spec, full_spec],
        out_specs=pl.BlockSpec((_R, _N), lambda i: (i, 0)),
        out_shape=jax.ShapeDtypeStruct((_N, _N), jnp.float32),
    )(n1p, n2p, n1p, n2p)
    return adj
